# baseline (speedup 1.0000x reference)
import functools

import jax
import jax.numpy as jnp
from jax import lax
from jax.experimental import pallas as pl
from jax.experimental.pallas import tpu as pltpu
from jax.experimental.pallas import tpu_sc as plsc

N = 10000
E = 320000
D = 128
DO = 128
RW = 16

NTILES = 32
RPT = 320
NPAD = NTILES * RPT
DUMMY = RPT

C2 = 512
NCH2 = E // C2

SCAN = 2560
SUB = 128
PCAP = 256

F32 = jnp.float32
I32 = jnp.int32

_mesh = plsc.VectorSubcoreMesh(core_axis_name="c", subcore_axis_name="s")


def _wid():
    return lax.axis_index("s") * 2 + lax.axis_index("c")


def _k2_body(a_hbm, posT_hbm, src_hbm, dst_hbm, mg_out, rel_out,
             srcb, dstb, mbuf, relbuf, pbuf, sem):
    w = _wid()
    nch = 19 + jnp.where(w < NCH2 - 19 * NTILES, 1, 0)
    lanes = lax.iota(I32, 16)
    zeros16f = jnp.zeros((16,), F32)

    pltpu.sync_copy(posT_hbm, pbuf)

    def zrow(r, carry):
        relbuf[pl.ds(r * 16, 16)] = zeros16f
        return carry

    lax.fori_loop(0, C2 * RW // 16, zrow, 0)

    def chunk(ci, carry):
        chunk_id = w + ci * NTILES
        base = chunk_id * C2
        cs = pltpu.async_copy(src_hbm.at[pl.ds(base, C2)], srcb, sem)
        cd = pltpu.async_copy(dst_hbm.at[pl.ds(base, C2)], dstb, sem)
        cs.wait()
        cd.wait()
        copies = []
        for j in range(C2 // SUB):
            sl = pl.ds(j * SUB, SUB)
            copies.append(pltpu.async_copy(
                a_hbm.at[srcb.at[sl]], mbuf.at[sl], sem))
        for v in range(C2 // 16):
            sid = srcb[pl.ds(v * 16, 16)]
            did = dstb[pl.ds(v * 16, 16)]
            rows = v * 16 + lanes
            for p in range(3):
                ps = plsc.load_gather(pbuf, [p * N + sid])
                pd = plsc.load_gather(pbuf, [p * N + did])
                plsc.store_scatter(relbuf, [rows * RW + p], ps - pd)
        for c in copies:
            c.wait()
        pltpu.sync_copy(mbuf, mg_out.at[pl.ds(base, C2)])
        pltpu.sync_copy(relbuf, rel_out.at[pl.ds(base * RW, C2 * RW)])
        return carry

    lax.fori_loop(0, nch, chunk, 0)


@functools.partial(
    pl.kernel,
    out_type=[
        jax.ShapeDtypeStruct((E, D), F32),
        jax.ShapeDtypeStruct((E * RW,), F32),
    ],
    mesh=_mesh,
    compiler_params=pltpu.CompilerParams(needs_layout_passes=False),
    scratch_types=[
        pltpu.VMEM((C2,), I32),
        pltpu.VMEM((C2,), I32),
        pltpu.VMEM((C2, D), F32),
        pltpu.VMEM((C2 * RW,), F32),
        pltpu.VMEM((3 * N,), F32),
        pltpu.SemaphoreType.DMA,
    ],
)
def _k2(a_hbm, posT_hbm, src_hbm, dst_hbm, mg_out, rel_out,
        srcb, dstb, mbuf, relbuf, pbuf, sem):
    _k2_body(a_hbm, posT_hbm, src_hbm, dst_hbm, mg_out, rel_out,
             srcb, dstb, mbuf, relbuf, pbuf, sem)


def _k3_body(mg_ref, rel_ref, w1_ref, b1_ref, wp_ref, w2_ref, b2_ref, h_ref):
    h1 = jnp.dot(mg_ref[...], w1_ref[...], preferred_element_type=F32) \
        + b1_ref[...] \
        + jnp.dot(rel_ref[...], wp_ref[...], preferred_element_type=F32)
    h1 = jnp.maximum(h1, 0.0)
    h = jnp.dot(h1, w2_ref[...], preferred_element_type=F32) + b2_ref[...]
    h_ref[...] = h.astype(jnp.bfloat16)


def _k3(mg, rel, W1x, b1r, Wp, W2, b2r):
    blk = 1280
    return pl.pallas_call(
        _k3_body,
        grid=(E // blk,),
        in_specs=[
            pl.BlockSpec((blk, D), lambda i: (i, 0)),
            pl.BlockSpec((blk, RW), lambda i: (i, 0)),
            pl.BlockSpec((D, D), lambda i: (0, 0)),
            pl.BlockSpec((1, D), lambda i: (0, 0)),
            pl.BlockSpec((RW, D), lambda i: (0, 0)),
            pl.BlockSpec((D, DO), lambda i: (0, 0)),
            pl.BlockSpec((1, DO), lambda i: (0, 0)),
        ],
        out_specs=pl.BlockSpec((blk, DO), lambda i: (i, 0)),
        out_shape=jax.ShapeDtypeStruct((E, DO), jnp.bfloat16),
    )(mg, rel, W1x, b1r, Wp, W2, b2r)


NOUT = E // SCAN


def _rowmax(rowb, sld, outl, par):
    base = par * SUB

    def row(r, carry):
        li = sld[pl.ds(base + r, 16)][0]
        orow = outl.at[li]
        for k in range(DO // 32):
            sl = pl.ds(k * 32, 32)
            orow[sl] = jnp.maximum(orow[sl], rowb[base + r, sl])
        return carry

    lax.fori_loop(0, SUB, row, 0)


def _flush(h_hbm, idp, ldp, sid, sld, rowb, outl, gsem, nb):
    par = nb % 2
    for j in range(SUB // 16):
        sl = pl.ds(j * 16, 16)
        sid[pl.ds(par * SUB + j * 16, 16)] = idp[sl]
        sld[pl.ds(par * SUB + j * 16, 16)] = ldp[sl]
        idp[sl] = idp[pl.ds(SUB + j * 16, 16)]
        ldp[sl] = ldp[pl.ds(SUB + j * 16, 16)]
    pltpu.async_copy(
        h_hbm.at[sid.at[pl.ds(par * SUB, SUB)]],
        rowb.at[pl.ds(par * SUB, SUB)], gsem.at[par])

    @pl.when(nb >= 1)
    def _():
        opar = 1 - par
        pltpu.make_async_copy(
            h_hbm.at[sid.at[pl.ds(opar * SUB, SUB)]],
            rowb.at[pl.ds(opar * SUB, SUB)], gsem.at[opar]).wait()
        _rowmax(rowb, sld, outl, opar)


def _k4_body(h_hbm, dst_hbm, out_hbm, dchunk, idp, ldp, sid, sld, rowb, outl,
             gsem, dsem):
    w = _wid()
    lo = w * RPT
    neg = jnp.full((32,), -jnp.inf, jnp.bfloat16)
    zeros16 = jnp.zeros((16,), I32)
    dummy16 = jnp.full((16,), DUMMY, I32)
    lanes = lax.iota(I32, 16)

    for j in range(PCAP // 16):
        sl = pl.ds(j * 16, 16)
        idp[sl] = zeros16
        ldp[sl] = dummy16

    def initrow(r, carry):
        orow = outl.at[r]
        for k in range(DO // 32):
            orow[pl.ds(k * 32, 32)] = neg
        return carry

    lax.fori_loop(0, RPT + 1, initrow, 0)

    def dfire(c):
        pltpu.async_copy(
            dst_hbm.at[pl.ds(c * SCAN, SCAN)],
            dchunk.at[pl.ds((c % 2) * SCAN, SCAN)], dsem.at[c % 2])

    def dwait(c):
        pltpu.make_async_copy(
            dst_hbm.at[pl.ds(c * SCAN, SCAN)],
            dchunk.at[pl.ds((c % 2) * SCAN, SCAN)], dsem.at[c % 2]).wait()

    dfire(0)

    def outer(c, carry):
        dwait(c)

        @pl.when(c + 1 < NOUT)
        def _():
            dfire(c + 1)

        dbase = (c % 2) * SCAN

        def inner(b, carry):
            off, nb = carry
            for i in range(SUB // 16):
                d = dchunk[pl.ds(dbase + b * SUB + i * 16, 16)]
                inr = (d >= lo) & (d < lo + RPT)
                eids = c * SCAN + b * SUB + i * 16 + lanes
                plsc.store_compressed(idp.at[pl.ds(off, 16)], eids, mask=inr)
                plsc.store_compressed(ldp.at[pl.ds(off, 16)], d - lo, mask=inr)
                off = off + plsc.all_reduce_population_count(inr)[0]

            def do_flush(carry):
                off, nb = carry
                _flush(h_hbm, idp, ldp, sid, sld, rowb, outl, gsem, nb)
                return (off - SUB, nb + 1)

            return lax.cond(off >= SUB, do_flush, lambda x: x, (off, nb))

        return lax.fori_loop(0, SCAN // SUB, inner, carry)

    off, nb = lax.fori_loop(0, NOUT, outer, (jnp.int32(0), jnp.int32(0)))
    _flush(h_hbm, idp, ldp, sid, sld, rowb, outl, gsem, nb)
    _flush(h_hbm, idp, ldp, sid, sld, rowb, outl, gsem, nb + 1)
    nb = nb + 2

    @pl.when(nb >= 1)
    def _():
        par = (nb - 1) % 2
        pltpu.make_async_copy(
            h_hbm.at[sid.at[pl.ds(par * SUB, SUB)]],
            rowb.at[pl.ds(par * SUB, SUB)], gsem.at[par]).wait()
        _rowmax(rowb, sld, outl, par)

    def fixrow(r, carry):
        orow = outl.at[r]
        for k in range(DO // 32):
            sl = pl.ds(k * 32, 32)
            v = orow[sl]
            orow[sl] = jnp.where(v == neg, jnp.zeros((32,), jnp.bfloat16), v)
        return carry

    lax.fori_loop(0, RPT, fixrow, 0)
    pltpu.sync_copy(outl.at[pl.ds(0, RPT)], out_hbm.at[pl.ds(lo, RPT)])


@functools.partial(
    pl.kernel,
    out_type=jax.ShapeDtypeStruct((NPAD, DO), jnp.bfloat16),
    mesh=_mesh,
    compiler_params=pltpu.CompilerParams(needs_layout_passes=False,
                                         use_tc_tiling_on_sc=False),
    scratch_types=[
        pltpu.VMEM((2 * SCAN,), I32),
        pltpu.VMEM((PCAP,), I32),
        pltpu.VMEM((PCAP,), I32),
        pltpu.VMEM((2 * SUB,), I32),
        pltpu.VMEM((2 * SUB + 16,), I32),
        pltpu.VMEM((2 * SUB, DO), jnp.bfloat16),
        pltpu.VMEM((RPT + 1, DO), jnp.bfloat16),
        pltpu.SemaphoreType.DMA((2,)),
        pltpu.SemaphoreType.DMA((2,)),
    ],
)
def _k4(h_hbm, dst_hbm, out_hbm, dchunk, idp, ldp, sid, sld, rowb, outl,
        gsem, dsem):
    _k4_body(h_hbm, dst_hbm, out_hbm, dchunk, idp, ldp, sid, sld, rowb, outl,
             gsem, dsem)


def kernel(x, pos, batch, edge_index, W1, b1, W2, b2):
    src = edge_index[0]
    dst = edge_index[1]
    posT = jnp.transpose(pos).reshape(-1)
    W1x = W1[:D]
    Wp = jnp.pad(W1[D:], ((0, RW - 3), (0, 0)))
    b1r = b1.reshape(1, D)
    b2r = b2.reshape(1, DO)

    mg, rel = _k2(x, posT, src, dst)
    h = _k3(mg, rel.reshape(E, RW), W1x, b1r, Wp, W2, b2r)
    out = _k4(h, dst)
    return (out[:N].astype(F32), pos, batch, edge_index)


# speedup vs baseline: 1.1567x; 1.1567x over previous
import functools

import jax
import jax.numpy as jnp
from jax import lax
from jax.experimental import pallas as pl
from jax.experimental.pallas import tpu as pltpu
from jax.experimental.pallas import tpu_sc as plsc

N = 10000
E = 320000
D = 128
DO = 128
RW = 16

NTILES = 32
RPT = 320
NPAD = NTILES * RPT
DUMMY = RPT

C2 = 512
NCH2 = E // C2

SCAN = 2560
SUB = 128
PCAP = 256

F32 = jnp.float32
I32 = jnp.int32

_mesh = plsc.VectorSubcoreMesh(core_axis_name="c", subcore_axis_name="s")


def _wid():
    return lax.axis_index("s") * 2 + lax.axis_index("c")


def _k1_body(x_ref, w_ref, b_ref, a_ref):
    a_ref[...] = jnp.dot(x_ref[...], w_ref[...],
                         preferred_element_type=F32) + b_ref[...]


def _k1(x, W1x, b1r):
    blk = 1000
    return pl.pallas_call(
        _k1_body,
        grid=(N // blk,),
        in_specs=[
            pl.BlockSpec((blk, D), lambda i: (i, 0)),
            pl.BlockSpec((D, D), lambda i: (0, 0)),
            pl.BlockSpec((1, D), lambda i: (0, 0)),
        ],
        out_specs=pl.BlockSpec((blk, D), lambda i: (i, 0)),
        out_shape=jax.ShapeDtypeStruct((N, D), F32),
    )(x, W1x, b1r)


def _k2_body(a_hbm, posT_hbm, src_hbm, dst_hbm, mg_out, rel_out,
             srcb, dstb, mbuf, relbuf, pbuf, sem):
    w = _wid()
    nch = 19 + jnp.where(w < NCH2 - 19 * NTILES, 1, 0)
    lanes = lax.iota(I32, 16)
    zeros16f = jnp.zeros((16,), F32)

    pltpu.sync_copy(posT_hbm, pbuf)

    def zrow(r, carry):
        relbuf[pl.ds(r * 16, 16)] = zeros16f
        return carry

    lax.fori_loop(0, C2 * RW // 16, zrow, 0)

    def chunk(ci, carry):
        chunk_id = w + ci * NTILES
        base = chunk_id * C2
        cs = pltpu.async_copy(src_hbm.at[pl.ds(base, C2)], srcb, sem)
        cd = pltpu.async_copy(dst_hbm.at[pl.ds(base, C2)], dstb, sem)
        cs.wait()
        cd.wait()
        copies = []
        for j in range(C2 // SUB):
            sl = pl.ds(j * SUB, SUB)
            copies.append(pltpu.async_copy(
                a_hbm.at[srcb.at[sl]], mbuf.at[sl], sem))
        for v in range(C2 // 16):
            sid = srcb[pl.ds(v * 16, 16)]
            did = dstb[pl.ds(v * 16, 16)]
            rows = v * 16 + lanes
            for p in range(3):
                ps = plsc.load_gather(pbuf, [p * N + sid])
                pd = plsc.load_gather(pbuf, [p * N + did])
                plsc.store_scatter(relbuf, [rows * RW + p], ps - pd)
        for c in copies:
            c.wait()
        pltpu.sync_copy(mbuf, mg_out.at[pl.ds(base, C2)])
        pltpu.sync_copy(relbuf, rel_out.at[pl.ds(base * RW, C2 * RW)])
        return carry

    lax.fori_loop(0, nch, chunk, 0)


@functools.partial(
    pl.kernel,
    out_type=[
        jax.ShapeDtypeStruct((E, D), F32),
        jax.ShapeDtypeStruct((E * RW,), F32),
    ],
    mesh=_mesh,
    compiler_params=pltpu.CompilerParams(needs_layout_passes=False),
    scratch_types=[
        pltpu.VMEM((C2,), I32),
        pltpu.VMEM((C2,), I32),
        pltpu.VMEM((C2, D), F32),
        pltpu.VMEM((C2 * RW,), F32),
        pltpu.VMEM((3 * N,), F32),
        pltpu.SemaphoreType.DMA,
    ],
)
def _k2(a_hbm, posT_hbm, src_hbm, dst_hbm, mg_out, rel_out,
        srcb, dstb, mbuf, relbuf, pbuf, sem):
    _k2_body(a_hbm, posT_hbm, src_hbm, dst_hbm, mg_out, rel_out,
             srcb, dstb, mbuf, relbuf, pbuf, sem)


def _k3_body(mg_ref, rel_ref, wp_ref, w2_ref, b2_ref, h_ref):
    h1 = mg_ref[...] + jnp.dot(rel_ref[...], wp_ref[...],
                               preferred_element_type=F32)
    h1 = jnp.maximum(h1, 0.0)
    h = jnp.dot(h1, w2_ref[...], preferred_element_type=F32) + b2_ref[...]
    h_ref[...] = h.astype(jnp.bfloat16)


def _k3(mg, rel, Wp, W2, b2r):
    blk = 1280
    return pl.pallas_call(
        _k3_body,
        grid=(E // blk,),
        in_specs=[
            pl.BlockSpec((blk, D), lambda i: (i, 0)),
            pl.BlockSpec((blk, RW), lambda i: (i, 0)),
            pl.BlockSpec((RW, D), lambda i: (0, 0)),
            pl.BlockSpec((D, DO), lambda i: (0, 0)),
            pl.BlockSpec((1, DO), lambda i: (0, 0)),
        ],
        out_specs=pl.BlockSpec((blk, DO), lambda i: (i, 0)),
        out_shape=jax.ShapeDtypeStruct((E, DO), jnp.bfloat16),
    )(mg, rel, Wp, W2, b2r)


NOUT = E // SCAN


def _rowmax(rowb, sld, outl, par):
    base = par * SUB

    def row(r2, carry):
        r = 2 * r2
        li0 = sld[pl.ds(base + r, 16)][0]
        li1 = sld[pl.ds(base + r + 1, 16)][0]
        o0 = outl.at[li0]
        o1 = outl.at[li1]
        v0 = [rowb[base + r, pl.ds(k * 32, 32)] for k in range(DO // 32)]
        v1 = [rowb[base + r + 1, pl.ds(k * 32, 32)] for k in range(DO // 32)]
        for k in range(DO // 32):
            sl = pl.ds(k * 32, 32)
            o0[sl] = jnp.maximum(o0[sl], v0[k])
        for k in range(DO // 32):
            sl = pl.ds(k * 32, 32)
            o1[sl] = jnp.maximum(o1[sl], v1[k])
        return carry

    lax.fori_loop(0, SUB // 2, row, 0)


def _flush(h_hbm, idp, ldp, sid, sld, rowb, outl, gsem, nb):
    par = nb % 2
    for j in range(SUB // 16):
        sl = pl.ds(j * 16, 16)
        sid[pl.ds(par * SUB + j * 16, 16)] = idp[sl]
        sld[pl.ds(par * SUB + j * 16, 16)] = ldp[sl]
        idp[sl] = idp[pl.ds(SUB + j * 16, 16)]
        ldp[sl] = ldp[pl.ds(SUB + j * 16, 16)]
    pltpu.async_copy(
        h_hbm.at[sid.at[pl.ds(par * SUB, SUB)]],
        rowb.at[pl.ds(par * SUB, SUB)], gsem.at[par])

    @pl.when(nb >= 1)
    def _():
        opar = 1 - par
        pltpu.make_async_copy(
            h_hbm.at[sid.at[pl.ds(opar * SUB, SUB)]],
            rowb.at[pl.ds(opar * SUB, SUB)], gsem.at[opar]).wait()
        _rowmax(rowb, sld, outl, opar)


def _k4_body(h_hbm, dst_hbm, out_hbm, dchunk, idp, ldp, sid, sld, rowb, outl,
             gsem, dsem):
    w = _wid()
    lo = w * RPT
    neg = jnp.full((32,), -jnp.inf, jnp.bfloat16)
    zeros16 = jnp.zeros((16,), I32)
    dummy16 = jnp.full((16,), DUMMY, I32)
    lanes = lax.iota(I32, 16)

    for j in range(PCAP // 16):
        sl = pl.ds(j * 16, 16)
        idp[sl] = zeros16
        ldp[sl] = dummy16

    def initrow(r, carry):
        orow = outl.at[r]
        for k in range(DO // 32):
            orow[pl.ds(k * 32, 32)] = neg
        return carry

    lax.fori_loop(0, RPT + 1, initrow, 0)

    def dfire(c):
        pltpu.async_copy(
            dst_hbm.at[pl.ds(c * SCAN, SCAN)],
            dchunk.at[pl.ds((c % 2) * SCAN, SCAN)], dsem.at[c % 2])

    def dwait(c):
        pltpu.make_async_copy(
            dst_hbm.at[pl.ds(c * SCAN, SCAN)],
            dchunk.at[pl.ds((c % 2) * SCAN, SCAN)], dsem.at[c % 2]).wait()

    dfire(0)

    def outer(c, carry):
        dwait(c)

        @pl.when(c + 1 < NOUT)
        def _():
            dfire(c + 1)

        dbase = (c % 2) * SCAN

        def inner(b, carry):
            off, nb = carry
            for i in range(SUB // 16):
                d = dchunk[pl.ds(dbase + b * SUB + i * 16, 16)]
                inr = (d >= lo) & (d < lo + RPT)
                eids = c * SCAN + b * SUB + i * 16 + lanes
                plsc.store_compressed(idp.at[pl.ds(off, 16)], eids, mask=inr)
                plsc.store_compressed(ldp.at[pl.ds(off, 16)], d - lo, mask=inr)
                off = off + plsc.all_reduce_population_count(inr)[0]

            def do_flush(carry):
                off, nb = carry
                _flush(h_hbm, idp, ldp, sid, sld, rowb, outl, gsem, nb)
                return (off - SUB, nb + 1)

            return lax.cond(off >= SUB, do_flush, lambda x: x, (off, nb))

        return lax.fori_loop(0, SCAN // SUB, inner, carry)

    off, nb = lax.fori_loop(0, NOUT, outer, (jnp.int32(0), jnp.int32(0)))
    _flush(h_hbm, idp, ldp, sid, sld, rowb, outl, gsem, nb)
    _flush(h_hbm, idp, ldp, sid, sld, rowb, outl, gsem, nb + 1)
    nb = nb + 2

    @pl.when(nb >= 1)
    def _():
        par = (nb - 1) % 2
        pltpu.make_async_copy(
            h_hbm.at[sid.at[pl.ds(par * SUB, SUB)]],
            rowb.at[pl.ds(par * SUB, SUB)], gsem.at[par]).wait()
        _rowmax(rowb, sld, outl, par)

    def fixrow(r, carry):
        orow = outl.at[r]
        for k in range(DO // 32):
            sl = pl.ds(k * 32, 32)
            v = orow[sl]
            orow[sl] = jnp.where(v == neg, jnp.zeros((32,), jnp.bfloat16), v)
        return carry

    lax.fori_loop(0, RPT, fixrow, 0)
    pltpu.sync_copy(outl.at[pl.ds(0, RPT)], out_hbm.at[pl.ds(lo, RPT)])


@functools.partial(
    pl.kernel,
    out_type=jax.ShapeDtypeStruct((NPAD, DO), jnp.bfloat16),
    mesh=_mesh,
    compiler_params=pltpu.CompilerParams(needs_layout_passes=False,
                                         use_tc_tiling_on_sc=False),
    scratch_types=[
        pltpu.VMEM((2 * SCAN,), I32),
        pltpu.VMEM((PCAP,), I32),
        pltpu.VMEM((PCAP,), I32),
        pltpu.VMEM((2 * SUB,), I32),
        pltpu.VMEM((2 * SUB + 16,), I32),
        pltpu.VMEM((2 * SUB, DO), jnp.bfloat16),
        pltpu.VMEM((RPT + 1, DO), jnp.bfloat16),
        pltpu.SemaphoreType.DMA((2,)),
        pltpu.SemaphoreType.DMA((2,)),
    ],
)
def _k4(h_hbm, dst_hbm, out_hbm, dchunk, idp, ldp, sid, sld, rowb, outl,
        gsem, dsem):
    _k4_body(h_hbm, dst_hbm, out_hbm, dchunk, idp, ldp, sid, sld, rowb, outl,
             gsem, dsem)


def kernel(x, pos, batch, edge_index, W1, b1, W2, b2):
    src = edge_index[0]
    dst = edge_index[1]
    posT = jnp.transpose(pos).reshape(-1)
    W1x = W1[:D]
    Wp = jnp.pad(W1[D:], ((0, RW - 3), (0, 0)))
    b1r = b1.reshape(1, D)
    b2r = b2.reshape(1, DO)

    a = _k1(x, W1x, b1r)
    mg, rel = _k2(a, posT, src, dst)
    h = _k3(mg, rel.reshape(E, RW), Wp, W2, b2r)
    out = _k4(h, dst)
    return (out[:N].astype(F32), pos, batch, edge_index)


# speedup vs baseline: 1.1831x; 1.0229x over previous
import functools

import jax
import jax.numpy as jnp
from jax import lax
from jax.experimental import pallas as pl
from jax.experimental.pallas import tpu as pltpu
from jax.experimental.pallas import tpu_sc as plsc

N = 10000
E = 320000
D = 128
DO = 128
RW = 16

NTILES = 32
RPT = 320
NPAD = NTILES * RPT
DUMMY = RPT

C2 = 512
NCH2 = E // C2

SCAN = 2560
SUB = 128
PCAP = 256

F32 = jnp.float32
I32 = jnp.int32

_mesh = plsc.VectorSubcoreMesh(core_axis_name="c", subcore_axis_name="s")


def _wid():
    return lax.axis_index("s") * 2 + lax.axis_index("c")


def _k1_body(x_ref, w_ref, b_ref, a_ref):
    a_ref[...] = jnp.dot(x_ref[...], w_ref[...],
                         preferred_element_type=F32) + b_ref[...]


def _k1(x, W1x, b1r):
    blk = 1000
    return pl.pallas_call(
        _k1_body,
        grid=(N // blk,),
        in_specs=[
            pl.BlockSpec((blk, D), lambda i: (i, 0)),
            pl.BlockSpec((D, D), lambda i: (0, 0)),
            pl.BlockSpec((1, D), lambda i: (0, 0)),
        ],
        out_specs=pl.BlockSpec((blk, D), lambda i: (i, 0)),
        out_shape=jax.ShapeDtypeStruct((N, D), F32),
    )(x, W1x, b1r)


def _k2_body(a_hbm, posT_hbm, src_hbm, dst_hbm, mg_out, rel_out,
             srcb, dstb, mbuf, relbuf, pbuf, sem):
    w = _wid()
    nch = 19 + jnp.where(w < NCH2 - 19 * NTILES, 1, 0)
    lanes = lax.iota(I32, 16)
    zeros16f = jnp.zeros((16,), F32)

    pltpu.sync_copy(posT_hbm, pbuf)

    def zrow(r, carry):
        relbuf[pl.ds(r * 16, 16)] = zeros16f
        return carry

    lax.fori_loop(0, C2 * RW // 16, zrow, 0)

    def chunk(ci, carry):
        chunk_id = w + ci * NTILES
        base = chunk_id * C2
        cs = pltpu.async_copy(src_hbm.at[pl.ds(base, C2)], srcb, sem)
        cd = pltpu.async_copy(dst_hbm.at[pl.ds(base, C2)], dstb, sem)
        cs.wait()
        cd.wait()
        copies = []
        for j in range(C2 // SUB):
            sl = pl.ds(j * SUB, SUB)
            copies.append(pltpu.async_copy(
                a_hbm.at[srcb.at[sl]], mbuf.at[sl], sem))
        for v in range(C2 // 16):
            sid = srcb[pl.ds(v * 16, 16)]
            did = dstb[pl.ds(v * 16, 16)]
            rows = v * 16 + lanes
            for p in range(3):
                ps = plsc.load_gather(pbuf, [p * N + sid])
                pd = plsc.load_gather(pbuf, [p * N + did])
                plsc.store_scatter(relbuf, [rows * RW + p], ps - pd)
        for c in copies:
            c.wait()
        pltpu.sync_copy(mbuf, mg_out.at[pl.ds(base, C2)])
        pltpu.sync_copy(relbuf, rel_out.at[pl.ds(base * RW, C2 * RW)])
        return carry

    lax.fori_loop(0, nch, chunk, 0)


@functools.partial(
    pl.kernel,
    out_type=[
        jax.ShapeDtypeStruct((E, D), F32),
        jax.ShapeDtypeStruct((E * RW,), F32),
    ],
    mesh=_mesh,
    compiler_params=pltpu.CompilerParams(needs_layout_passes=False),
    scratch_types=[
        pltpu.VMEM((C2,), I32),
        pltpu.VMEM((C2,), I32),
        pltpu.VMEM((C2, D), F32),
        pltpu.VMEM((C2 * RW,), F32),
        pltpu.VMEM((3 * N,), F32),
        pltpu.SemaphoreType.DMA,
    ],
)
def _k2(a_hbm, posT_hbm, src_hbm, dst_hbm, mg_out, rel_out,
        srcb, dstb, mbuf, relbuf, pbuf, sem):
    _k2_body(a_hbm, posT_hbm, src_hbm, dst_hbm, mg_out, rel_out,
             srcb, dstb, mbuf, relbuf, pbuf, sem)


def _k3_body(mg_ref, rel_ref, wp_ref, w2_ref, b2_ref, h_ref):
    h1 = mg_ref[...] + jnp.dot(rel_ref[...], wp_ref[...],
                               preferred_element_type=F32)
    h1 = jnp.maximum(h1, 0.0)
    h = jnp.dot(h1, w2_ref[...], preferred_element_type=F32) + b2_ref[...]
    h_ref[...] = h.astype(jnp.bfloat16)


def _k3(mg, rel, Wp, W2, b2r):
    blk = 1280
    return pl.pallas_call(
        _k3_body,
        grid=(E // blk,),
        in_specs=[
            pl.BlockSpec((blk, D), lambda i: (i, 0)),
            pl.BlockSpec((blk, RW), lambda i: (i, 0)),
            pl.BlockSpec((RW, D), lambda i: (0, 0)),
            pl.BlockSpec((D, DO), lambda i: (0, 0)),
            pl.BlockSpec((1, DO), lambda i: (0, 0)),
        ],
        out_specs=pl.BlockSpec((blk, DO), lambda i: (i, 0)),
        out_shape=jax.ShapeDtypeStruct((E, DO), jnp.bfloat16),
    )(mg, rel, Wp, W2, b2r)


NOUT = E // SCAN


def _rowmax(rowb, sld, outl, par):
    base = par * SUB

    def row(r4, carry):
        r = 4 * r4
        lis = [sld[pl.ds(base + r + j, 16)][0] for j in range(4)]
        orows = [outl.at[li] for li in lis]
        vs = [[rowb[base + r + j, pl.ds(k * 32, 32)] for k in range(DO // 32)]
              for j in range(4)]
        for j in range(4):
            for k in range(DO // 32):
                sl = pl.ds(k * 32, 32)
                orows[j][sl] = jnp.maximum(orows[j][sl], vs[j][k])
        return carry

    lax.fori_loop(0, SUB // 4, row, 0)


def _flush(h_hbm, idp, ldp, sid, sld, rowb, outl, gsem, nb):
    par = nb % 2
    for j in range(SUB // 16):
        sl = pl.ds(j * 16, 16)
        sid[pl.ds(par * SUB + j * 16, 16)] = idp[sl]
        sld[pl.ds(par * SUB + j * 16, 16)] = ldp[sl]
        idp[sl] = idp[pl.ds(SUB + j * 16, 16)]
        ldp[sl] = ldp[pl.ds(SUB + j * 16, 16)]
    pltpu.async_copy(
        h_hbm.at[sid.at[pl.ds(par * SUB, SUB)]],
        rowb.at[pl.ds(par * SUB, SUB)], gsem.at[par])

    @pl.when(nb >= 1)
    def _():
        opar = 1 - par
        pltpu.make_async_copy(
            h_hbm.at[sid.at[pl.ds(opar * SUB, SUB)]],
            rowb.at[pl.ds(opar * SUB, SUB)], gsem.at[opar]).wait()
        _rowmax(rowb, sld, outl, opar)


def _k4_body(h_hbm, dst_hbm, out_hbm, dchunk, idp, ldp, sid, sld, rowb, outl,
             gsem, dsem):
    w = _wid()
    lo = w * RPT
    neg = jnp.full((32,), -jnp.inf, jnp.bfloat16)
    zeros16 = jnp.zeros((16,), I32)
    dummy16 = jnp.full((16,), DUMMY, I32)
    lanes = lax.iota(I32, 16)

    for j in range(PCAP // 16):
        sl = pl.ds(j * 16, 16)
        idp[sl] = zeros16
        ldp[sl] = dummy16

    def initrow(r, carry):
        orow = outl.at[r]
        for k in range(DO // 32):
            orow[pl.ds(k * 32, 32)] = neg
        return carry

    lax.fori_loop(0, RPT + 1, initrow, 0)

    def dfire(c):
        pltpu.async_copy(
            dst_hbm.at[pl.ds(c * SCAN, SCAN)],
            dchunk.at[pl.ds((c % 2) * SCAN, SCAN)], dsem.at[c % 2])

    def dwait(c):
        pltpu.make_async_copy(
            dst_hbm.at[pl.ds(c * SCAN, SCAN)],
            dchunk.at[pl.ds((c % 2) * SCAN, SCAN)], dsem.at[c % 2]).wait()

    dfire(0)

    def outer(c, carry):
        dwait(c)

        @pl.when(c + 1 < NOUT)
        def _():
            dfire(c + 1)

        dbase = (c % 2) * SCAN

        def inner(b, carry):
            off, nb = carry
            for i in range(SUB // 16):
                d = dchunk[pl.ds(dbase + b * SUB + i * 16, 16)]
                inr = (d >= lo) & (d < lo + RPT)
                eids = c * SCAN + b * SUB + i * 16 + lanes
                plsc.store_compressed(idp.at[pl.ds(off, 16)], eids, mask=inr)
                plsc.store_compressed(ldp.at[pl.ds(off, 16)], d - lo, mask=inr)
                off = off + plsc.all_reduce_population_count(inr)[0]

            def do_flush(carry):
                off, nb = carry
                _flush(h_hbm, idp, ldp, sid, sld, rowb, outl, gsem, nb)
                return (off - SUB, nb + 1)

            return lax.cond(off >= SUB, do_flush, lambda x: x, (off, nb))

        return lax.fori_loop(0, SCAN // SUB, inner, carry)

    off, nb = lax.fori_loop(0, NOUT, outer, (jnp.int32(0), jnp.int32(0)))
    _flush(h_hbm, idp, ldp, sid, sld, rowb, outl, gsem, nb)
    _flush(h_hbm, idp, ldp, sid, sld, rowb, outl, gsem, nb + 1)
    nb = nb + 2

    @pl.when(nb >= 1)
    def _():
        par = (nb - 1) % 2
        pltpu.make_async_copy(
            h_hbm.at[sid.at[pl.ds(par * SUB, SUB)]],
            rowb.at[pl.ds(par * SUB, SUB)], gsem.at[par]).wait()
        _rowmax(rowb, sld, outl, par)

    def fixrow(r, carry):
        orow = outl.at[r]
        for k in range(DO // 32):
            sl = pl.ds(k * 32, 32)
            v = orow[sl]
            orow[sl] = jnp.where(v == neg, jnp.zeros((32,), jnp.bfloat16), v)
        return carry

    lax.fori_loop(0, RPT, fixrow, 0)
    pltpu.sync_copy(outl.at[pl.ds(0, RPT)], out_hbm.at[pl.ds(lo, RPT)])


@functools.partial(
    pl.kernel,
    out_type=jax.ShapeDtypeStruct((NPAD, DO), jnp.bfloat16),
    mesh=_mesh,
    compiler_params=pltpu.CompilerParams(needs_layout_passes=False,
                                         use_tc_tiling_on_sc=False),
    scratch_types=[
        pltpu.VMEM((2 * SCAN,), I32),
        pltpu.VMEM((PCAP,), I32),
        pltpu.VMEM((PCAP,), I32),
        pltpu.VMEM((2 * SUB,), I32),
        pltpu.VMEM((2 * SUB + 16,), I32),
        pltpu.VMEM((2 * SUB, DO), jnp.bfloat16),
        pltpu.VMEM((RPT + 1, DO), jnp.bfloat16),
        pltpu.SemaphoreType.DMA((2,)),
        pltpu.SemaphoreType.DMA((2,)),
    ],
)
def _k4(h_hbm, dst_hbm, out_hbm, dchunk, idp, ldp, sid, sld, rowb, outl,
        gsem, dsem):
    _k4_body(h_hbm, dst_hbm, out_hbm, dchunk, idp, ldp, sid, sld, rowb, outl,
             gsem, dsem)


def kernel(x, pos, batch, edge_index, W1, b1, W2, b2):
    src = edge_index[0]
    dst = edge_index[1]
    posT = jnp.transpose(pos).reshape(-1)
    W1x = W1[:D]
    Wp = jnp.pad(W1[D:], ((0, RW - 3), (0, 0)))
    b1r = b1.reshape(1, D)
    b2r = b2.reshape(1, DO)

    a = _k1(x, W1x, b1r)
    mg, rel = _k2(a, posT, src, dst)
    h = _k3(mg, rel.reshape(E, RW), Wp, W2, b2r)
    out = _k4(h, dst)
    return (out[:N].astype(F32), pos, batch, edge_index)


# speedup vs baseline: 1.3297x; 1.1239x over previous
import functools

import jax
import jax.numpy as jnp
from jax import lax
from jax.experimental import pallas as pl
from jax.experimental.pallas import tpu as pltpu
from jax.experimental.pallas import tpu_sc as plsc

N = 10000
E = 320000
D = 128
DO = 128
RW = 16

NTILES = 32
RPT = 320
NPAD = NTILES * RPT
DUMMY = RPT

C2 = 512
NCH2 = E // C2

SCAN = 2560
SUB = 128
PCAP = 256

F32 = jnp.float32
I32 = jnp.int32

_mesh = plsc.VectorSubcoreMesh(core_axis_name="c", subcore_axis_name="s")


def _wid():
    return lax.axis_index("s") * 2 + lax.axis_index("c")


def _k1_body(x_ref, w_ref, b_ref, a_ref):
    a_ref[...] = jnp.dot(x_ref[...], w_ref[...],
                         preferred_element_type=F32) + b_ref[...]


def _k1(x, W1x, b1r):
    blk = 1000
    return pl.pallas_call(
        _k1_body,
        grid=(N // blk,),
        in_specs=[
            pl.BlockSpec((blk, D), lambda i: (i, 0)),
            pl.BlockSpec((D, D), lambda i: (0, 0)),
            pl.BlockSpec((1, D), lambda i: (0, 0)),
        ],
        out_specs=pl.BlockSpec((blk, D), lambda i: (i, 0)),
        out_shape=jax.ShapeDtypeStruct((N, D), F32),
    )(x, W1x, b1r)


def _k2_body(a_hbm, posT_hbm, src_hbm, dst_hbm, mg_out, rel_out,
             srcb, dstb, mbuf, relbuf, pbuf, sem):
    w = _wid()
    nch = 19 + jnp.where(w < NCH2 - 19 * NTILES, 1, 0)
    lanes = lax.iota(I32, 16)
    zeros16f = jnp.zeros((16,), F32)

    pltpu.sync_copy(posT_hbm, pbuf)

    def zrow(r, carry):
        relbuf[pl.ds(r * 16, 16)] = zeros16f
        return carry

    lax.fori_loop(0, C2 * RW // 16, zrow, 0)

    def chunk(ci, carry):
        chunk_id = w + ci * NTILES
        base = chunk_id * C2
        cs = pltpu.async_copy(src_hbm.at[pl.ds(base, C2)], srcb, sem)
        cd = pltpu.async_copy(dst_hbm.at[pl.ds(base, C2)], dstb, sem)
        cs.wait()
        cd.wait()
        copies = []
        for j in range(C2 // SUB):
            sl = pl.ds(j * SUB, SUB)
            copies.append(pltpu.async_copy(
                a_hbm.at[srcb.at[sl]], mbuf.at[sl], sem))
        for v in range(C2 // 16):
            sid = srcb[pl.ds(v * 16, 16)]
            did = dstb[pl.ds(v * 16, 16)]
            rows = v * 16 + lanes
            for p in range(3):
                ps = plsc.load_gather(pbuf, [p * N + sid])
                pd = plsc.load_gather(pbuf, [p * N + did])
                plsc.store_scatter(relbuf, [rows * RW + p], ps - pd)
        for c in copies:
            c.wait()
        pltpu.sync_copy(mbuf, mg_out.at[pl.ds(base, C2)])
        pltpu.sync_copy(relbuf, rel_out.at[pl.ds(base * RW, C2 * RW)])
        return carry

    lax.fori_loop(0, nch, chunk, 0)


@functools.partial(
    pl.kernel,
    out_type=[
        jax.ShapeDtypeStruct((E, D), F32),
        jax.ShapeDtypeStruct((E * RW,), F32),
    ],
    mesh=_mesh,
    compiler_params=pltpu.CompilerParams(needs_layout_passes=False),
    scratch_types=[
        pltpu.VMEM((C2,), I32),
        pltpu.VMEM((C2,), I32),
        pltpu.VMEM((C2, D), F32),
        pltpu.VMEM((C2 * RW,), F32),
        pltpu.VMEM((3 * N,), F32),
        pltpu.SemaphoreType.DMA,
    ],
)
def _k2(a_hbm, posT_hbm, src_hbm, dst_hbm, mg_out, rel_out,
        srcb, dstb, mbuf, relbuf, pbuf, sem):
    _k2_body(a_hbm, posT_hbm, src_hbm, dst_hbm, mg_out, rel_out,
             srcb, dstb, mbuf, relbuf, pbuf, sem)


def _k3_body(mg_ref, rel_ref, wp_ref, w2_ref, b2_ref, h_ref):
    h1 = mg_ref[...] + jnp.dot(rel_ref[...], wp_ref[...],
                               preferred_element_type=F32)
    h1 = jnp.maximum(h1, 0.0)
    h = jnp.dot(h1, w2_ref[...], preferred_element_type=F32) + b2_ref[...]
    h_ref[...] = h.astype(jnp.bfloat16)


def _k3(mg, rel, Wp, W2, b2r):
    blk = 1280
    return pl.pallas_call(
        _k3_body,
        grid=(E // blk,),
        in_specs=[
            pl.BlockSpec((blk, D), lambda i: (i, 0)),
            pl.BlockSpec((blk, RW), lambda i: (i, 0)),
            pl.BlockSpec((RW, D), lambda i: (0, 0)),
            pl.BlockSpec((D, DO), lambda i: (0, 0)),
            pl.BlockSpec((1, DO), lambda i: (0, 0)),
        ],
        out_specs=pl.BlockSpec((blk, DO), lambda i: (i, 0)),
        out_shape=jax.ShapeDtypeStruct((E, DO), jnp.bfloat16),
    )(mg, rel, Wp, W2, b2r)


NOUT = E // SCAN


def _rowmax(rowb, sld, outl, par):
    base = par * SUB

    def row(r4, carry):
        r = 4 * r4
        lis = [sld[pl.ds(base + r + j, 16)][0] for j in range(4)]
        orows = [outl.at[li] for li in lis]
        vs = [[rowb[base + r + j, pl.ds(k * 32, 32)] for k in range(DO // 32)]
              for j in range(4)]
        for j in range(4):
            for k in range(DO // 32):
                sl = pl.ds(k * 32, 32)
                orows[j][sl] = jnp.maximum(orows[j][sl], vs[j][k])
        return carry

    lax.fori_loop(0, SUB // 4, row, 0)


def _flush(h_hbm, idp, ldp, sid, sld, rowb, outl, gsem, nb):
    par = nb % 2
    for j in range(SUB // 16):
        sl = pl.ds(j * 16, 16)
        sid[pl.ds(par * SUB + j * 16, 16)] = idp[sl]
        sld[pl.ds(par * SUB + j * 16, 16)] = ldp[sl]
        idp[sl] = idp[pl.ds(SUB + j * 16, 16)]
        ldp[sl] = ldp[pl.ds(SUB + j * 16, 16)]
    pltpu.async_copy(
        h_hbm.at[sid.at[pl.ds(par * SUB, SUB)]],
        rowb.at[pl.ds(par * SUB, SUB)], gsem.at[par])

    @pl.when(nb >= 1)
    def _():
        opar = 1 - par
        pltpu.make_async_copy(
            h_hbm.at[sid.at[pl.ds(opar * SUB, SUB)]],
            rowb.at[pl.ds(opar * SUB, SUB)], gsem.at[opar]).wait()
        _rowmax(rowb, sld, outl, opar)


def _k4_body(h_hbm, dst_hbm, out_hbm, dchunk, idp, ldp, sid, sld, rowb, outl,
             gsem, dsem):
    w = _wid()
    lo = w * RPT
    neg = jnp.full((32,), -jnp.inf, jnp.bfloat16)
    zeros16 = jnp.zeros((16,), I32)
    dummy16 = jnp.full((16,), DUMMY, I32)
    lanes = lax.iota(I32, 16)

    for j in range(PCAP // 16):
        sl = pl.ds(j * 16, 16)
        idp[sl] = zeros16
        ldp[sl] = dummy16

    def initrow(r, carry):
        orow = outl.at[r]
        for k in range(DO // 32):
            orow[pl.ds(k * 32, 32)] = neg
        return carry

    lax.fori_loop(0, RPT + 1, initrow, 0)

    def dfire(c):
        pltpu.async_copy(
            dst_hbm.at[pl.ds(c * SCAN, SCAN)],
            dchunk.at[pl.ds((c % 2) * SCAN, SCAN)], dsem.at[c % 2])

    def dwait(c):
        pltpu.make_async_copy(
            dst_hbm.at[pl.ds(c * SCAN, SCAN)],
            dchunk.at[pl.ds((c % 2) * SCAN, SCAN)], dsem.at[c % 2]).wait()

    dfire(0)

    def outer(c, carry):
        dwait(c)

        @pl.when(c + 1 < NOUT)
        def _():
            dfire(c + 1)

        dbase = (c % 2) * SCAN

        def inner(b, carry):
            off, nb = carry
            dv = [dchunk[pl.ds(dbase + b * SUB + i * 16, 16)]
                  for i in range(SUB // 16)]
            masks = [(d >= lo) & (d < lo + RPT) for d in dv]
            cnts = [plsc.all_reduce_population_count(m)[0] for m in masks]
            offs = []
            for i in range(SUB // 16):
                offs.append(off)
                off = off + cnts[i]
            for i in range(SUB // 16):
                eids = c * SCAN + b * SUB + i * 16 + lanes
                plsc.store_compressed(idp.at[pl.ds(offs[i], 16)], eids,
                                      mask=masks[i])
                plsc.store_compressed(ldp.at[pl.ds(offs[i], 16)], dv[i] - lo,
                                      mask=masks[i])

            def do_flush(carry):
                off, nb = carry
                _flush(h_hbm, idp, ldp, sid, sld, rowb, outl, gsem, nb)
                return (off - SUB, nb + 1)

            return lax.cond(off >= SUB, do_flush, lambda x: x, (off, nb))

        return lax.fori_loop(0, SCAN // SUB, inner, carry)

    off, nb = lax.fori_loop(0, NOUT, outer, (jnp.int32(0), jnp.int32(0)))
    _flush(h_hbm, idp, ldp, sid, sld, rowb, outl, gsem, nb)
    _flush(h_hbm, idp, ldp, sid, sld, rowb, outl, gsem, nb + 1)
    nb = nb + 2

    @pl.when(nb >= 1)
    def _():
        par = (nb - 1) % 2
        pltpu.make_async_copy(
            h_hbm.at[sid.at[pl.ds(par * SUB, SUB)]],
            rowb.at[pl.ds(par * SUB, SUB)], gsem.at[par]).wait()
        _rowmax(rowb, sld, outl, par)

    def fixrow(r, carry):
        orow = outl.at[r]
        for k in range(DO // 32):
            sl = pl.ds(k * 32, 32)
            v = orow[sl]
            orow[sl] = jnp.where(v == neg, jnp.zeros((32,), jnp.bfloat16), v)
        return carry

    lax.fori_loop(0, RPT, fixrow, 0)
    pltpu.sync_copy(outl.at[pl.ds(0, RPT)], out_hbm.at[pl.ds(lo, RPT)])


@functools.partial(
    pl.kernel,
    out_type=jax.ShapeDtypeStruct((NPAD, DO), jnp.bfloat16),
    mesh=_mesh,
    compiler_params=pltpu.CompilerParams(needs_layout_passes=False,
                                         use_tc_tiling_on_sc=False),
    scratch_types=[
        pltpu.VMEM((2 * SCAN,), I32),
        pltpu.VMEM((PCAP,), I32),
        pltpu.VMEM((PCAP,), I32),
        pltpu.VMEM((2 * SUB,), I32),
        pltpu.VMEM((2 * SUB + 16,), I32),
        pltpu.VMEM((2 * SUB, DO), jnp.bfloat16),
        pltpu.VMEM((RPT + 1, DO), jnp.bfloat16),
        pltpu.SemaphoreType.DMA((2,)),
        pltpu.SemaphoreType.DMA((2,)),
    ],
)
def _k4(h_hbm, dst_hbm, out_hbm, dchunk, idp, ldp, sid, sld, rowb, outl,
        gsem, dsem):
    _k4_body(h_hbm, dst_hbm, out_hbm, dchunk, idp, ldp, sid, sld, rowb, outl,
             gsem, dsem)


def kernel(x, pos, batch, edge_index, W1, b1, W2, b2):
    src = edge_index[0]
    dst = edge_index[1]
    posT = jnp.transpose(pos).reshape(-1)
    W1x = W1[:D]
    Wp = jnp.pad(W1[D:], ((0, RW - 3), (0, 0)))
    b1r = b1.reshape(1, D)
    b2r = b2.reshape(1, DO)

    a = _k1(x, W1x, b1r)
    mg, rel = _k2(a, posT, src, dst)
    h = _k3(mg, rel.reshape(E, RW), Wp, W2, b2r)
    out = _k4(h, dst)
    return (out[:N].astype(F32), pos, batch, edge_index)
